# Initial kernel scaffold; baseline (speedup 1.0000x reference)
import jax
import jax.numpy as jnp
from jax.experimental import pallas as pl


def kernel(edge_index, des, tweet, num_prop, cat_prop, W_des, b_des, W_tw, b_tw, W_np, b_np, W_cp, b_cp, Wb1, bb1, Wb2, bb2, Wb3, bb3, Wi, bi, Wg1, as1, ad1, bg1, Wg2, as2, ad2, bg2, Wo1, bo1, Wa1, ba1, Wa2, ba2, Wa3, ba3, Wo2, bo2):
    raise NotImplementedError("write your pallas kernel here")


# speedup vs baseline: 12.1986x; 12.1986x over previous
import functools

import jax
import jax.numpy as jnp
import numpy as np
from jax import lax
from jax.experimental import pallas as pl
from jax.experimental.pallas import tpu as pltpu
from jax.experimental.pallas import tpu_sc as plsc

N = 10000
E = 320000
RB = 1000
GRID = N // RB
FH = 64
ACC_W = 72
EPT = E // 16
CB = 1000
CH = 16
RPT = N // 16

_f32 = jnp.float32


def _lrelu01(v):
    return jnp.where(v >= 0, v, 0.01 * v)


def _lrelu02(v):
    return jnp.where(v >= 0, v, 0.2 * v)


def _front_body(des_r, tw_r, np_r, cp_r,
                Wdes_r, bdes_r, Wtw_r, btw_r, Wnp_r, bnp_r, Wcp_r, bcp_r,
                Wb1a_r, Wb1b_r, Wb1c_r, Wb1d_r, bb1_r,
                Wb2_r, bb2_r, Wb3_r, bb3_r, Wi_r, bi_r, Wg1_r, A1_r,
                h_o, att_o):
    dot = functools.partial(jnp.dot, preferred_element_type=_f32)
    d = _lrelu01(dot(des_r[...], Wdes_r[...]) + bdes_r[...])
    t = _lrelu01(dot(tw_r[...], Wtw_r[...]) + btw_r[...])
    p = _lrelu01(dot(np_r[...], Wnp_r[...]) + bnp_r[...])
    c = _lrelu01(dot(cp_r[...], Wcp_r[...]) + bcp_r[...])
    x = jnp.maximum(dot(d, Wb1a_r[...]) + dot(t, Wb1b_r[...])
                    + dot(p, Wb1c_r[...]) + dot(c, Wb1d_r[...]) + bb1_r[...], 0)
    x = jnp.maximum(dot(x, Wb2_r[...]) + bb2_r[...], 0)
    x = dot(x, Wb3_r[...]) + bb3_r[...]
    xi = _lrelu01(dot(x, Wi_r[...]) + bi_r[...])
    h = dot(xi, Wg1_r[...])
    h_o[...] = h
    att_o[...] = dot(h, A1_r[...])


def _run_front(des, tweet, nprop, cprop, weights):
    (Wdes, bdes, Wtw, btw, Wnp, bnp, Wcp, bcp,
     Wb1a, Wb1b, Wb1c, Wb1d, bb1, Wb2, bb2, Wb3, bb3, Wi, bi, Wg1, A1) = weights
    full = lambda a: pl.BlockSpec(a.shape, lambda i: (0,) * a.ndim)
    row = lambda w: pl.BlockSpec((RB, w), lambda i: (i, 0))
    in_specs = [row(768), row(768), row(5), row(3)] + [full(w) for w in weights]
    out_specs = [pl.BlockSpec((RB, 128), lambda i: (i, 0)),
                 pl.BlockSpec((RB, 8), lambda i: (i, 0))]
    out_shape = [jax.ShapeDtypeStruct((N, 128), _f32),
                 jax.ShapeDtypeStruct((N, 8), _f32)]
    return pl.pallas_call(
        _front_body, grid=(GRID,), in_specs=in_specs, out_specs=out_specs,
        out_shape=out_shape)(des, tweet, nprop, cprop, *weights)


def _make_gat_edges(H):
    C = 128 // H
    mesh = plsc.VectorSubcoreMesh(core_axis_name="c", subcore_axis_name="s",
                                  num_cores=2, num_subcores=16)

    @functools.partial(
        pl.kernel, mesh=mesh,
        compiler_params=pltpu.CompilerParams(use_tc_tiling_on_sc=False,
                                             needs_layout_passes=False),
        out_type=jax.ShapeDtypeStruct((2, N, ACC_W), _f32),
        scratch_types=[
            pltpu.VMEM((CB,), jnp.int32),
            pltpu.VMEM((CB,), jnp.int32),
            pltpu.VMEM((N, 2 * H), _f32),
            pltpu.VMEM((CH, FH), _f32),
            pltpu.VMEM((CH, ACC_W), _f32),
            pltpu.VMEM((16,), _f32),
            pltpu.VMEM_SHARED((N, ACC_W), _f32),
            pltpu.SemaphoreType.DMA,
        ])
    def gat_edges(src_hbm, dst_hbm, att_hbm, hA_hbm, hB_hbm, zeros_hbm, b_hbm,
                  out_hbm, src_v, dst_v, att_v, rbuf, wrows, bv, acc_sh, sem):
        c = lax.axis_index("c")
        s = lax.axis_index("s")
        pltpu.sync_copy(zeros_hbm.at[pl.ds(s * RPT, RPT)],
                        acc_sh.at[pl.ds(s * RPT, RPT)])
        pltpu.sync_copy(att_hbm, att_v)
        pltpu.sync_copy(b_hbm, bv)
        iota = lax.iota(jnp.int32, 16)
        zero16 = jnp.zeros((16,), _f32)
        for f in range(FH + H, ACC_W):
            plsc.store_scatter(wrows, [iota, jnp.full((16,), f, jnp.int32)],
                               zero16)
        Bv = bv[...]
        plsc.subcore_barrier()

        def outer(oi, ocarry):
            pltpu.sync_copy(src_hbm.at[pl.ds(s * EPT + oi * CB, CB)], src_v)
            pltpu.sync_copy(dst_hbm.at[pl.ds(s * EPT + oi * CB, CB)], dst_v)

            def body(ci, carry):
                base = ci * CH
                src16 = src_v[pl.ds(base, CH)]
                dst16 = dst_v[pl.ds(base, CH)]
                exs = []
                for h in range(H):
                    a_s = plsc.load_gather(
                        att_v, [src16, jnp.full((16,), h, jnp.int32)])
                    a_d = plsc.load_gather(
                        att_v, [dst16, jnp.full((16,), H + h, jnp.int32)])
                    ex = jnp.exp(_lrelu02(a_s + a_d) - Bv)
                    exs.append(ex)
                    plsc.store_scatter(
                        wrows, [iota, jnp.full((16,), FH + h, jnp.int32)], ex)

                def do_half(cs, h_hbm):
                    pltpu.async_copy(h_hbm.at[src16], rbuf, sem).wait()
                    for f in range(FH):
                        fv = jnp.full((16,), f, jnp.int32)
                        col = plsc.load_gather(rbuf, [iota, fv])
                        plsc.store_scatter(wrows, [iota, fv],
                                           col * exs[(cs * FH + f) // C])

                @pl.when(c == 0)
                def _():
                    do_half(0, hA_hbm)

                @pl.when(c == 1)
                def _():
                    do_half(1, hB_hbm)

                pltpu.sync_copy(wrows, acc_sh.at[dst16], add=True)
                return carry

            lax.fori_loop(0, CB // CH, body, 0)
            return ocarry

        lax.fori_loop(0, EPT // CB, outer, 0)
        plsc.subcore_barrier()
        pltpu.sync_copy(acc_sh.at[pl.ds(s * RPT, RPT)],
                        out_hbm.at[c, pl.ds(s * RPT, RPT)])

    return gat_edges


_make_gat_edges = functools.lru_cache(maxsize=None)(_make_gat_edges)


def _mid_body(p0_r, p1_r, h1_r, att1_r, b_r, bg1_r, Wg2_r, A2_r, Rep_r,
              h2_o, att2_o):
    dot = functools.partial(jnp.dot, preferred_element_type=_f32)
    acc0 = p0_r[...]
    acc1 = p1_r[...]
    att = att1_r[...]
    B1 = b_r[0, 0]
    exs = jnp.exp(_lrelu02(att[:, 0:4] + att[:, 4:8]) - B1)
    Rep = Rep_r[...]
    edge_num = jnp.concatenate([acc0[:, :FH], acc1[:, :FH]], axis=1)
    numer = edge_num + dot(exs, Rep) * h1_r[...]
    den = acc0[:, FH:FH + 4] + exs
    out1 = numer / (dot(den, Rep) + 1e-16) + bg1_r[...]
    x2 = out1 + out1
    h2 = dot(x2, Wg2_r[...])
    h2_o[...] = h2
    att2_o[...] = dot(h2, A2_r[...])


def _run_mid(p0, p1, h1, att1, b1, bg1, Wg2, A2, Rep):
    full = lambda a: pl.BlockSpec(a.shape, lambda i: (0,) * a.ndim)
    row = lambda w: pl.BlockSpec((RB, w), lambda i: (i, 0))
    in_specs = [row(ACC_W), row(ACC_W), row(128), row(8),
                full(b1), full(bg1), full(Wg2), full(A2), full(Rep)]
    out_specs = [pl.BlockSpec((RB, 128), lambda i: (i, 0)),
                 pl.BlockSpec((RB, 8), lambda i: (i, 0))]
    out_shape = [jax.ShapeDtypeStruct((N, 128), _f32),
                 jax.ShapeDtypeStruct((N, 8), _f32)]
    return pl.pallas_call(
        _mid_body, grid=(GRID,), in_specs=in_specs, out_specs=out_specs,
        out_shape=out_shape)(p0, p1, h1, att1, b1, bg1, Wg2, A2, Rep)


def _back_body(p0_r, p1_r, h2_r, att2_r, b_r,
               bg2_r, Wo1_r, bo1_r, Wa1_r, ba1_r, Wa2_r, ba2_r,
               Wa3_r, ba3_r, Wo2_r, bo2_r, y_o):
    dot = functools.partial(jnp.dot, preferred_element_type=_f32)
    acc0 = p0_r[...]
    acc1 = p1_r[...]
    att = att2_r[...]
    B2 = b_r[0, 0]
    ex = jnp.exp(_lrelu02(att[:, 0:1] + att[:, 4:5]) - B2)
    edge_num = jnp.concatenate([acc0[:, :FH], acc1[:, :FH]], axis=1)
    numer = edge_num + ex * h2_r[...]
    den = acc0[:, FH:FH + 1] + ex + 1e-16
    out2 = numer / den + bg2_r[...]
    x = _lrelu01(dot(out2, Wo1_r[...]) + bo1_r[...])
    x = x + x
    x = jnp.maximum(dot(x, Wa1_r[...]) + ba1_r[...], 0)
    x = jnp.maximum(dot(x, Wa2_r[...]) + ba2_r[...], 0)
    x = dot(x, Wa3_r[...]) + ba3_r[...]
    y_o[...] = dot(x, Wo2_r[...]) + bo2_r[...]


def _run_back(p0, p1, h2, att2, b2, weights):
    full = lambda a: pl.BlockSpec(a.shape, lambda i: (0,) * a.ndim)
    row = lambda w: pl.BlockSpec((RB, w), lambda i: (i, 0))
    in_specs = [row(ACC_W), row(ACC_W), row(128), row(8), full(b2)] + \
        [full(w) for w in weights]
    out_specs = pl.BlockSpec((RB, 2), lambda i: (i, 0))
    out_shape = jax.ShapeDtypeStruct((N, 2), _f32)
    return pl.pallas_call(
        _back_body, grid=(GRID,), in_specs=in_specs, out_specs=out_specs,
        out_shape=out_shape)(p0, p1, h2, att2, b2, *weights)


def kernel(edge_index, des, tweet, num_prop, cat_prop,
           W_des, b_des, W_tw, b_tw, W_np, b_np, W_cp, b_cp,
           Wb1, bb1, Wb2, bb2, Wb3, bb3, Wi, bi,
           Wg1, as1, ad1, bg1, Wg2, as2, ad2, bg2,
           Wo1, bo1, Wa1, ba1, Wa2, ba2, Wa3, ba3, Wo2, bo2):
    src = edge_index[0]
    dst = edge_index[1]
    r1 = lambda b: b.reshape(1, -1)

    A_s1 = jax.scipy.linalg.block_diag(*[as1[h][:, None] for h in range(4)])
    A_d1 = jax.scipy.linalg.block_diag(*[ad1[h][:, None] for h in range(4)])
    A1 = jnp.concatenate([A_s1, A_d1], axis=1)
    A2 = jnp.concatenate([as2.T] * 4 + [ad2.T] * 4, axis=1)
    Rep = jnp.asarray(np.repeat(np.eye(4, dtype=np.float32), 32, axis=1))

    front_w = (W_des, r1(b_des), W_tw, r1(b_tw), W_np, r1(b_np), W_cp,
               r1(b_cp), Wb1[0:32], Wb1[32:64], Wb1[64:96], Wb1[96:128],
               r1(bb1), Wb2, r1(bb2), Wb3, r1(bb3), Wi, r1(bi), Wg1, A1)
    h1, att1 = _run_front(des, tweet, num_prop, cat_prop, front_w)

    B1 = jnp.maximum(jnp.max(att1[:, 0:4]) + jnp.max(att1[:, 4:8]), 0.0)
    zeros = jnp.zeros((N, ACC_W), _f32)
    parts1 = _make_gat_edges(4)(src, dst, att1, h1[:, :FH], h1[:, FH:],
                                zeros, jnp.full((16,), B1, _f32))
    h2, att2 = _run_mid(parts1[0], parts1[1], h1, att1,
                        B1.reshape(1, 1), r1(bg1), Wg2, A2, Rep)

    B2 = jnp.maximum(jnp.max(att2[:, 0]) + jnp.max(att2[:, 4]), 0.0)
    parts2 = _make_gat_edges(4)(src, dst, att2, h2[:, :FH],
                                h2[:, FH:], zeros, jnp.full((16,), B2, _f32))
    back_w = (r1(bg2), Wo1, r1(bo1), Wa1, r1(ba1), Wa2, r1(ba2),
              Wa3, r1(ba3), Wo2, r1(bo2))
    return _run_back(parts2[0], parts2[1], h2, att2, B2.reshape(1, 1), back_w)


# speedup vs baseline: 15.8071x; 1.2958x over previous
import functools

import jax
import jax.numpy as jnp
import numpy as np
from jax import lax
from jax.experimental import pallas as pl
from jax.experimental.pallas import tpu as pltpu
from jax.experimental.pallas import tpu_sc as plsc

N = 10000
E = 320000
RB = 1000
GRID = N // RB
FH = 64
ACC_W = 72
EPT = E // 16
CB = 800
CH = 16
RPT = N // 16

_f32 = jnp.float32


def _lrelu01(v):
    return jnp.where(v >= 0, v, 0.01 * v)


def _lrelu02(v):
    return jnp.where(v >= 0, v, 0.2 * v)


def _front_body(des_r, tw_r, np_r, cp_r,
                Wdes_r, bdes_r, Wtw_r, btw_r, Wnp_r, bnp_r, Wcp_r, bcp_r,
                Wb1_r, bb1_r,
                Wb2_r, bb2_r, Wb3_r, bb3_r, Wi_r, bi_r, Wg1_r, A1_r,
                h_o, att_o):
    dot = functools.partial(jnp.dot, preferred_element_type=_f32)
    hdot = functools.partial(jnp.dot, preferred_element_type=_f32,
                             precision=jax.lax.Precision.HIGHEST)
    d = _lrelu01(dot(des_r[...], Wdes_r[...]) + bdes_r[...])
    t = _lrelu01(dot(tw_r[...], Wtw_r[...]) + btw_r[...])
    p = _lrelu01(dot(np_r[...], Wnp_r[...]) + bnp_r[...])
    c = _lrelu01(dot(cp_r[...], Wcp_r[...]) + bcp_r[...])
    xcat = jnp.concatenate([d, t, p, c], axis=1)
    x = jnp.maximum(dot(xcat, Wb1_r[...]) + bb1_r[...], 0)
    x = jnp.maximum(dot(x, Wb2_r[...]) + bb2_r[...], 0)
    x = dot(x, Wb3_r[...]) + bb3_r[...]
    xi = _lrelu01(dot(x, Wi_r[...]) + bi_r[...])
    h = dot(xi, Wg1_r[...])
    h_o[...] = h
    att_o[...] = hdot(h, A1_r[...])


def _run_front(des, tweet, nprop, cprop, weights):
    (Wdes, bdes, Wtw, btw, Wnp, bnp, Wcp, bcp,
     Wb1, bb1, Wb2, bb2, Wb3, bb3, Wi, bi, Wg1, A1) = weights
    full = lambda a: pl.BlockSpec(a.shape, lambda i: (0,) * a.ndim)
    row = lambda w: pl.BlockSpec((RB, w), lambda i: (i, 0))
    in_specs = [row(768), row(768), row(5), row(3)] + [full(w) for w in weights]
    out_specs = [pl.BlockSpec((RB, 128), lambda i: (i, 0)),
                 pl.BlockSpec((RB, 8), lambda i: (i, 0))]
    out_shape = [jax.ShapeDtypeStruct((N, 128), _f32),
                 jax.ShapeDtypeStruct((N, 8), _f32)]
    return pl.pallas_call(
        _front_body, grid=(GRID,), in_specs=in_specs, out_specs=out_specs,
        out_shape=out_shape)(des, tweet, nprop, cprop, *weights)


def _make_gat_edges(H):
    C = 128 // H
    mesh = plsc.VectorSubcoreMesh(core_axis_name="c", subcore_axis_name="s",
                                  num_cores=2, num_subcores=16)

    @functools.partial(
        pl.kernel, mesh=mesh,
        compiler_params=pltpu.CompilerParams(use_tc_tiling_on_sc=False,
                                             needs_layout_passes=False),
        out_type=jax.ShapeDtypeStruct((2, N, ACC_W), _f32),
        scratch_types=[
            pltpu.VMEM((CB,), jnp.int32),
            pltpu.VMEM((CB,), jnp.int32),
            pltpu.VMEM((N, 2 * H), _f32),
            pltpu.VMEM((CH, FH), _f32),
            pltpu.VMEM((CH, FH), _f32),
            pltpu.VMEM((CH, ACC_W), _f32),
            pltpu.VMEM((CH, ACC_W), _f32),
            pltpu.VMEM((16,), _f32),
            pltpu.VMEM_SHARED((N, ACC_W), _f32),
            pltpu.SemaphoreType.DMA,
            pltpu.SemaphoreType.DMA,
            pltpu.SemaphoreType.DMA,
            pltpu.SemaphoreType.DMA,
        ])
    def gat_edges(src_hbm, dst_hbm, att_hbm, hA_hbm, hB_hbm, b_hbm,
                  out_hbm, src_v, dst_v, att_v, rbufA, rbufB, wrowsA, wrowsB,
                  bv, acc_sh, gsA, gsB, ssA, ssB):
        c = lax.axis_index("c")
        s = lax.axis_index("s")
        pltpu.sync_copy(att_hbm, att_v)
        pltpu.sync_copy(b_hbm, bv)
        iota = lax.iota(jnp.int32, 16)
        zero16 = jnp.zeros((16,), _f32)
        zrow = jnp.zeros((16,), jnp.int32)
        def zf(f, zcarry):
            fv = jnp.full((16,), f, jnp.int32)
            plsc.store_scatter(wrowsA, [iota, fv], zero16)
            plsc.store_scatter(wrowsB, [iota, fv], zero16)
            return zcarry
        lax.fori_loop(0, ACC_W, zf, 0)
        def zloop(k, zcarry):
            pltpu.sync_copy(wrowsA, acc_sh.at[pl.ds(s * RPT + k * 16, 16)])
            return zcarry
        lax.fori_loop(0, RPT // 16, zloop, 0)
        pltpu.sync_copy(wrowsA, acc_sh.at[pl.ds(s * RPT + RPT - 16, 16)])
        Bv = bv[...]
        plsc.subcore_barrier()
        pltpu.async_copy(wrowsA, acc_sh.at[zrow], ssA, add=True)
        pltpu.async_copy(wrowsB, acc_sh.at[zrow], ssB, add=True)

        def gather(s16, rb, gs):
            @pl.when(c == 0)
            def _():
                pltpu.async_copy(hA_hbm.at[s16], rb, gs)

            @pl.when(c == 1)
            def _():
                pltpu.async_copy(hB_hbm.at[s16], rb, gs)

        def attn(s16, d16, wr, ss):
            pltpu.make_async_copy(wr, acc_sh.at[d16], ss).wait()
            exs = []
            for h in range(H):
                a_s = plsc.load_gather(
                    att_v, [s16, jnp.full((16,), h, jnp.int32)])
                a_d = plsc.load_gather(
                    att_v, [d16, jnp.full((16,), H + h, jnp.int32)])
                ex = jnp.exp(_lrelu02(a_s + a_d) - Bv)
                exs.append(ex)
                plsc.store_scatter(
                    wr, [iota, jnp.full((16,), FH + h, jnp.int32)], ex)
            return exs

        def outer(oi, ocarry):
            pltpu.sync_copy(src_hbm.at[pl.ds(s * EPT + oi * CB, CB)], src_v)
            pltpu.sync_copy(dst_hbm.at[pl.ds(s * EPT + oi * CB, CB)], dst_v)

            def pair(pi, carry):
                base = pi * 2 * CH
                s16a = src_v[pl.ds(base, CH)]
                d16a = dst_v[pl.ds(base, CH)]
                s16b = src_v[pl.ds(base + CH, CH)]
                d16b = dst_v[pl.ds(base + CH, CH)]
                gather(s16a, rbufA, gsA)
                gather(s16b, rbufB, gsB)
                exsA = attn(s16a, d16a, wrowsA, ssA)
                exsB = attn(s16b, d16b, wrowsB, ssB)

                CL = min(C, FH)

                def weight_issue(cs, h_hbm):
                    for rb, gs, wr, ss, s16, d16, exs in (
                            (rbufA, gsA, wrowsA, ssA, s16a, d16a, exsA),
                            (rbufB, gsB, wrowsB, ssB, s16b, d16b, exsB)):
                        pltpu.make_async_copy(h_hbm.at[s16], rb, gs).wait()
                        for k in range(FH // CL):
                            exk = exs[(cs * FH) // C + k]
                            kc = k * CL

                            def wbody(f, wcarry):
                                fv = jnp.full((16,), kc, jnp.int32) + f
                                col = plsc.load_gather(rb, [iota, fv])
                                plsc.store_scatter(wr, [iota, fv], col * exk)
                                return wcarry
                            lax.fori_loop(0, CL, wbody, 0)
                        pltpu.async_copy(wr, acc_sh.at[d16], ss, add=True)

                @pl.when(c == 0)
                def _():
                    weight_issue(0, hA_hbm)

                @pl.when(c == 1)
                def _():
                    weight_issue(1, hB_hbm)

                return carry

            lax.fori_loop(0, CB // (2 * CH), pair, 0)
            return ocarry

        lax.fori_loop(0, EPT // CB, outer, 0)
        pltpu.make_async_copy(wrowsA, acc_sh.at[zrow], ssA).wait()
        pltpu.make_async_copy(wrowsB, acc_sh.at[zrow], ssB).wait()
        plsc.subcore_barrier()

        def wloop(k, wcarry):
            pltpu.sync_copy(acc_sh.at[pl.ds(s * RPT + k * 16, 16)],
                            out_hbm.at[c, pl.ds(s * RPT + k * 16, 16)])
            return wcarry
        lax.fori_loop(0, RPT // 16, wloop, 0)
        pltpu.sync_copy(acc_sh.at[pl.ds(s * RPT + RPT - 16, 16)],
                        out_hbm.at[c, pl.ds(s * RPT + RPT - 16, 16)])

    return gat_edges


_make_gat_edges = functools.lru_cache(maxsize=None)(_make_gat_edges)


def _mid_body(p0_r, p1_r, h1_r, att1_r, b_r, bg1_r, Wg2_r, A2_r, Rep_r,
              h2_o, att2_o):
    dot = functools.partial(jnp.dot, preferred_element_type=_f32)
    hdot = functools.partial(jnp.dot, preferred_element_type=_f32,
                             precision=jax.lax.Precision.HIGHEST)
    acc0 = p0_r[...]
    acc1 = p1_r[...]
    att = att1_r[...]
    B1 = b_r[0, 0]
    exs = jnp.exp(_lrelu02(att[:, 0:4] + att[:, 4:8]) - B1)
    Rep = Rep_r[...]
    edge_num = jnp.concatenate([acc0[:, :FH], acc1[:, :FH]], axis=1)
    numer = edge_num + hdot(exs, Rep) * h1_r[...]
    den = acc0[:, FH:FH + 4] + exs
    out1 = numer / (hdot(den, Rep) + 1e-16) + bg1_r[...]
    x2 = out1 + out1
    h2 = dot(x2, Wg2_r[...])
    h2_o[...] = h2
    att2_o[...] = hdot(h2, A2_r[...])


def _run_mid(p0, p1, h1, att1, b1, bg1, Wg2, A2, Rep):
    full = lambda a: pl.BlockSpec(a.shape, lambda i: (0,) * a.ndim)
    row = lambda w: pl.BlockSpec((RB, w), lambda i: (i, 0))
    in_specs = [row(ACC_W), row(ACC_W), row(128), row(8),
                full(b1), full(bg1), full(Wg2), full(A2), full(Rep)]
    out_specs = [pl.BlockSpec((RB, 128), lambda i: (i, 0)),
                 pl.BlockSpec((RB, 8), lambda i: (i, 0))]
    out_shape = [jax.ShapeDtypeStruct((N, 128), _f32),
                 jax.ShapeDtypeStruct((N, 8), _f32)]
    return pl.pallas_call(
        _mid_body, grid=(GRID,), in_specs=in_specs, out_specs=out_specs,
        out_shape=out_shape)(p0, p1, h1, att1, b1, bg1, Wg2, A2, Rep)


def _back_body(p0_r, p1_r, h2_r, att2_r, b_r,
               bg2_r, Wo1_r, bo1_r, Wa1_r, ba1_r, Wa2_r, ba2_r,
               Wa3_r, ba3_r, Wo2_r, bo2_r, y_o):
    dot = functools.partial(jnp.dot, preferred_element_type=_f32)
    hdot = functools.partial(jnp.dot, preferred_element_type=_f32,
                             precision=jax.lax.Precision.HIGHEST)
    acc0 = p0_r[...]
    acc1 = p1_r[...]
    att = att2_r[...]
    B2 = b_r[0, 0]
    ex = jnp.exp(_lrelu02(att[:, 0:1] + att[:, 4:5]) - B2)
    edge_num = jnp.concatenate([acc0[:, :FH], acc1[:, :FH]], axis=1)
    numer = edge_num + ex * h2_r[...]
    den = acc0[:, FH:FH + 1] + ex + 1e-16
    out2 = numer / den + bg2_r[...]
    x = _lrelu01(dot(out2, Wo1_r[...]) + bo1_r[...])
    x = x + x
    x = jnp.maximum(dot(x, Wa1_r[...]) + ba1_r[...], 0)
    x = jnp.maximum(dot(x, Wa2_r[...]) + ba2_r[...], 0)
    x = dot(x, Wa3_r[...]) + ba3_r[...]
    y_o[...] = dot(x, Wo2_r[...]) + bo2_r[...]


def _run_back(p0, p1, h2, att2, b2, weights):
    full = lambda a: pl.BlockSpec(a.shape, lambda i: (0,) * a.ndim)
    row = lambda w: pl.BlockSpec((RB, w), lambda i: (i, 0))
    in_specs = [row(ACC_W), row(ACC_W), row(128), row(8), full(b2)] + \
        [full(w) for w in weights]
    out_specs = pl.BlockSpec((RB, 2), lambda i: (i, 0))
    out_shape = jax.ShapeDtypeStruct((N, 2), _f32)
    return pl.pallas_call(
        _back_body, grid=(GRID,), in_specs=in_specs, out_specs=out_specs,
        out_shape=out_shape)(p0, p1, h2, att2, b2, *weights)


def kernel(edge_index, des, tweet, num_prop, cat_prop,
           W_des, b_des, W_tw, b_tw, W_np, b_np, W_cp, b_cp,
           Wb1, bb1, Wb2, bb2, Wb3, bb3, Wi, bi,
           Wg1, as1, ad1, bg1, Wg2, as2, ad2, bg2,
           Wo1, bo1, Wa1, ba1, Wa2, ba2, Wa3, ba3, Wo2, bo2):
    src = edge_index[0]
    dst = edge_index[1]
    r1 = lambda b: b.reshape(1, -1)

    A_s1 = jax.scipy.linalg.block_diag(*[as1[h][:, None] for h in range(4)])
    A_d1 = jax.scipy.linalg.block_diag(*[ad1[h][:, None] for h in range(4)])
    A1 = jnp.concatenate([A_s1, A_d1], axis=1)
    A2 = jnp.concatenate([as2.T] * 4 + [ad2.T] * 4, axis=1)
    Rep = jnp.asarray(np.repeat(np.eye(4, dtype=np.float32), 32, axis=1))

    front_w = (W_des, r1(b_des), W_tw, r1(b_tw), W_np, r1(b_np), W_cp,
               r1(b_cp), Wb1,
               r1(bb1), Wb2, r1(bb2), Wb3, r1(bb3), Wi, r1(bi), Wg1, A1)
    h1, att1 = _run_front(des, tweet, num_prop, cat_prop, front_w)

    B1 = jnp.maximum(jnp.max(att1[:, 0:4]) + jnp.max(att1[:, 4:8]), 0.0)
    parts1 = _make_gat_edges(4)(src, dst, att1, h1[:, :FH], h1[:, FH:],
                                jnp.full((16,), B1, _f32))
    h2, att2 = _run_mid(parts1[0], parts1[1], h1, att1,
                        B1.reshape(1, 1), r1(bg1), Wg2, A2, Rep)

    B2 = jnp.maximum(jnp.max(att2[:, 0]) + jnp.max(att2[:, 4]), 0.0)
    parts2 = _make_gat_edges(4)(src, dst, att2, h2[:, :FH],
                                h2[:, FH:], jnp.full((16,), B2, _f32))
    back_w = (r1(bg2), Wo1, r1(bo1), Wa1, r1(ba1), Wa2, r1(ba2),
              Wa3, r1(ba3), Wo2, r1(bo2))
    return _run_back(parts2[0], parts2[1], h2, att2, B2.reshape(1, 1), back_w)
